# baseline (speedup 1.0000x reference)
import functools

import jax
import jax.numpy as jnp
from jax import lax
from jax.experimental import pallas as pl
from jax.experimental.pallas import tpu as pltpu
from jax.experimental.pallas import tpu_sc as plsc

BATCH = 16384
D = 32
L = 16
NC = 2
NS = 16
NW = NC * NS
BPW = BATCH // NW
GROUPS = BPW // L

_mesh = plsc.VectorSubcoreMesh(core_axis_name="c", subcore_axis_name="s")


@functools.partial(
    pl.kernel,
    mesh=_mesh,
    out_type=jax.ShapeDtypeStruct((BATCH,), jnp.float32),
    compiler_params=pltpu.CompilerParams(
        needs_layout_passes=False, use_tc_tiling_on_sc=False
    ),
    scratch_types=[
        pltpu.VMEM((BPW,), jnp.int32),
        pltpu.VMEM((BPW,), jnp.int32),
        pltpu.VMEM((BPW, D), jnp.float32),
        pltpu.VMEM((BPW, D), jnp.float32),
        pltpu.VMEM((BPW,), jnp.float32),
        pltpu.VMEM((L,), jnp.float32),
        pltpu.SemaphoreType.DMA,
        pltpu.SemaphoreType.DMA,
    ],
)
def _emb_fwd(uidx_hbm, midx_hbm, u_hbm, m_hbm, fc_hbm, out_hbm,
             uidx_v, midx_v, urows_v, mrows_v, out_v, fc_v, sem_u, sem_m):
    wid = lax.axis_index("s") * NC + lax.axis_index("c")
    base = wid * BPW

    pltpu.sync_copy(uidx_hbm.at[pl.ds(base, BPW)], uidx_v)
    pltpu.sync_copy(midx_hbm.at[pl.ds(base, BPW)], midx_v)
    cp_u = pltpu.async_copy(u_hbm.at[uidx_v], urows_v, sem_u)
    cp_m = pltpu.async_copy(m_hbm.at[midx_v], mrows_v, sem_m)
    pltpu.sync_copy(fc_hbm, fc_v)
    cp_u.wait()
    cp_m.wait()

    fcvec = fc_v[:]
    w = fcvec[0]
    b = fcvec[1]
    iota = lax.iota(jnp.int32, L)

    def group_body(g, _):
        rows = g * L + iota
        acc = jnp.zeros((L,), jnp.float32)
        for d in range(D):
            cols = (iota + d) & (D - 1)
            uv = plsc.load_gather(urows_v, [rows, cols])
            mv = plsc.load_gather(mrows_v, [rows, cols])
            acc = acc + uv * mv
        z = acc * w + b
        out_v[pl.ds(g * L, L)] = 1.0 / (1.0 + jnp.exp(-z))
        return 0

    lax.fori_loop(0, GROUPS, group_body, 0)

    pltpu.sync_copy(out_v, out_hbm.at[pl.ds(base, BPW)])


def kernel(x, u_table, m_table, fc_w, fc_b):
    uidx = x[:, 0].astype(jnp.int32)
    midx = x[:, 1].astype(jnp.int32)
    fc = jnp.zeros((L,), jnp.float32)
    fc = fc.at[0].set(fc_w[0, 0]).at[1].set(fc_b[0])
    out = _emb_fwd(uidx, midx, u_table, m_table, fc)
    return out.reshape(BATCH, 1)


# speedup vs baseline: 1.3777x; 1.3777x over previous
import functools

import jax
import jax.numpy as jnp
from jax import lax
from jax.experimental import pallas as pl
from jax.experimental.pallas import tpu as pltpu
from jax.experimental.pallas import tpu_sc as plsc

BATCH = 16384
D = 32
L = 16
NC = 2
NS = 16
NW = NC * NS
BPW = BATCH // NW
GROUPS = BPW // L
NBUF = 2
ROUNDS = GROUPS // NBUF

_mesh = plsc.VectorSubcoreMesh(core_axis_name="c", subcore_axis_name="s")


@functools.partial(
    pl.kernel,
    mesh=_mesh,
    out_type=jax.ShapeDtypeStruct((BATCH,), jnp.float32),
    compiler_params=pltpu.CompilerParams(
        needs_layout_passes=False, use_tc_tiling_on_sc=True
    ),
    scratch_types=[
        pltpu.VMEM((BPW,), jnp.int32),
        pltpu.VMEM((BPW,), jnp.int32),
        pltpu.VMEM((NBUF, L, 8, D), jnp.float32),
        pltpu.VMEM((NBUF, L, 8, D), jnp.float32),
        pltpu.VMEM((BPW,), jnp.float32),
        pltpu.VMEM((L,), jnp.float32),
        pltpu.SemaphoreType.DMA((NBUF,)),
    ],
)
def _emb_fwd(uidx_hbm, midx_hbm, u_hbm, m_hbm, fc_hbm, out_hbm,
             uidx_v, midx_v, ublk_v, mblk_v, out_v, fc_v, sem):
    wid = lax.axis_index("s") * NC + lax.axis_index("c")
    base = wid * BPW

    pltpu.sync_copy(uidx_hbm.at[pl.ds(base, BPW)], uidx_v)
    pltpu.sync_copy(midx_hbm.at[pl.ds(base, BPW)], midx_v)
    pltpu.sync_copy(fc_hbm, fc_v)

    fcvec = fc_v[:]
    w = fcvec[0]
    b = fcvec[1]
    iota = lax.iota(jnp.int32, L)

    def issue(g, slot):
        uvec = uidx_v[pl.ds(g * L, L)]
        mvec = midx_v[pl.ds(g * L, L)]
        for k in range(L):
            iu = uvec[k]
            im = mvec[k]
            pltpu.async_copy(
                u_hbm.at[pl.ds((iu >> 3) * 8, 8), :],
                ublk_v.at[slot, k], sem.at[slot])
            pltpu.async_copy(
                m_hbm.at[pl.ds((im >> 3) * 8, 8), :],
                mblk_v.at[slot, k], sem.at[slot])

    def drain(slot):
        for k in range(L):
            pltpu.make_async_copy(
                u_hbm.at[pl.ds(0, 8), :], ublk_v.at[slot, k],
                sem.at[slot]).wait()
            pltpu.make_async_copy(
                m_hbm.at[pl.ds(0, 8), :], mblk_v.at[slot, k],
                sem.at[slot]).wait()

    def compute(g, slot):
        uvec = uidx_v[pl.ds(g * L, L)]
        mvec = midx_v[pl.ds(g * L, L)]
        su = uvec & 7
        sm = mvec & 7
        acc = jnp.zeros((L,), jnp.float32)
        for d in range(D):
            cols = (iota + d) & (D - 1)
            uv = plsc.load_gather(ublk_v.at[slot], [iota, su, cols])
            mv = plsc.load_gather(mblk_v.at[slot], [iota, sm, cols])
            acc = acc + uv * mv
        z = acc * w + b
        out_v[pl.ds(g * L, L)] = 1.0 / (1.0 + jnp.exp(-z))

    for s in range(NBUF):
        issue(s, s)

    def round_body(r, _):
        for s in range(NBUF):
            g = r * NBUF + s
            drain(s)
            compute(g, s)

            @pl.when(r < ROUNDS - 1)
            def _():
                issue(g + NBUF, s)

        return 0

    lax.fori_loop(0, ROUNDS, round_body, 0)

    pltpu.sync_copy(out_v, out_hbm.at[pl.ds(base, BPW)])


def kernel(x, u_table, m_table, fc_w, fc_b):
    uidx = x[:, 0].astype(jnp.int32)
    midx = x[:, 1].astype(jnp.int32)
    fc = jnp.zeros((L,), jnp.float32)
    fc = fc.at[0].set(fc_w[0, 0]).at[1].set(fc_b[0])
    out = _emb_fwd(uidx, midx, u_table, m_table, fc)
    return out.reshape(BATCH, 1)


# speedup vs baseline: 3.2405x; 2.3520x over previous
import functools

import jax
import jax.numpy as jnp
from jax import lax
from jax.experimental import pallas as pl
from jax.experimental.pallas import tpu as pltpu
from jax.experimental.pallas import tpu_sc as plsc

BATCH = 16384
D = 32
L = 16
NC = 2
NS = 16
NW = NC * NS
BPW = BATCH // NW
NCH = 2
CHUNKS = BPW // NCH
NBUF = 2
ROUNDS = CHUNKS // NBUF
GROUPS = BPW // L

_mesh = plsc.VectorSubcoreMesh(core_axis_name="c", subcore_axis_name="s")


@functools.partial(
    pl.kernel,
    mesh=_mesh,
    out_type=jax.ShapeDtypeStruct((BATCH,), jnp.float32),
    compiler_params=pltpu.CompilerParams(
        needs_layout_passes=False, use_tc_tiling_on_sc=True
    ),
    scratch_types=[
        pltpu.VMEM((BPW + L,), jnp.int32),
        pltpu.VMEM((BPW + L,), jnp.int32),
        pltpu.VMEM((NBUF, NCH, 4, 8, 128), jnp.float32),
        pltpu.VMEM((NBUF, NCH, 4, 8, 128), jnp.float32),
        pltpu.VMEM((BPW, L), jnp.float32),
        pltpu.VMEM((BPW,), jnp.float32),
        pltpu.VMEM((L,), jnp.float32),
        pltpu.SemaphoreType.DMA((NBUF,)),
    ],
)
def _emb_fwd(uidx_hbm, midx_hbm, ut_hbm, mt_hbm, fc_hbm, out_hbm,
             uidx_v, midx_v, uslab_v, mslab_v, q_v, out_v, fc_v, sem):
    wid = lax.axis_index("s") * NC + lax.axis_index("c")
    base = wid * BPW

    pltpu.sync_copy(uidx_hbm.at[pl.ds(base, BPW)], uidx_v.at[pl.ds(0, BPW)])
    pltpu.sync_copy(midx_hbm.at[pl.ds(base, BPW)], midx_v.at[pl.ds(0, BPW)])
    pltpu.sync_copy(fc_hbm, fc_v)

    fcvec = fc_v[:]
    w = fcvec[0]
    b = fcvec[1]
    iota = lax.iota(jnp.int32, L)
    tr_lo = iota >> 3
    tr_hi = tr_lo + 2
    sub = iota & 7

    def issue(g, slot):
        uvec = uidx_v[pl.ds(g * NCH, L)]
        mvec = midx_v[pl.ds(g * NCH, L)]
        for k in range(NCH):
            cu = pl.multiple_of((uvec[k] >> 7) * 128, 128)
            cm = pl.multiple_of((mvec[k] >> 7) * 128, 128)
            for tr in range(4):
                pltpu.async_copy(
                    ut_hbm.at[pl.ds(tr * 8, 8), pl.ds(cu, 128)],
                    uslab_v.at[slot, k, tr], sem.at[slot])
                pltpu.async_copy(
                    mt_hbm.at[pl.ds(tr * 8, 8), pl.ds(cm, 128)],
                    mslab_v.at[slot, k, tr], sem.at[slot])

    def drain(slot):
        for k in range(NCH):
            for tr in range(4):
                pltpu.make_async_copy(
                    ut_hbm.at[pl.ds(0, 8), pl.ds(0, 128)],
                    uslab_v.at[slot, k, tr], sem.at[slot]).wait()
                pltpu.make_async_copy(
                    mt_hbm.at[pl.ds(0, 8), pl.ds(0, 128)],
                    mslab_v.at[slot, k, tr], sem.at[slot]).wait()

    def compute(g, slot):
        uvec = uidx_v[pl.ds(g * NCH, L)]
        mvec = midx_v[pl.ds(g * NCH, L)]
        for k in range(NCH):
            lu = jnp.full((L,), uvec[k] & 127, jnp.int32)
            lm = jnp.full((L,), mvec[k] & 127, jnp.int32)
            ulo = plsc.load_gather(uslab_v.at[slot, k], [tr_lo, sub, lu])
            uhi = plsc.load_gather(uslab_v.at[slot, k], [tr_hi, sub, lu])
            mlo = plsc.load_gather(mslab_v.at[slot, k], [tr_lo, sub, lm])
            mhi = plsc.load_gather(mslab_v.at[slot, k], [tr_hi, sub, lm])
            q_v[g * NCH + k] = ulo * mlo + uhi * mhi

    for s in range(NBUF):
        issue(s, s)

    def round_body(r, _):
        for s in range(NBUF):
            g = r * NBUF + s
            drain(s)
            compute(g, s)

            @pl.when(g + NBUF < CHUNKS)
            def _():
                issue(g + NBUF, s)

        return 0

    lax.fori_loop(0, ROUNDS, round_body, 0)

    def reduce_body(g, _):
        rows = g * L + iota
        acc = jnp.zeros((L,), jnp.float32)
        for j in range(L):
            cols = (iota + j) & (L - 1)
            acc = acc + plsc.load_gather(q_v, [rows, cols])
        z = acc * w + b
        out_v[pl.ds(g * L, L)] = 1.0 / (1.0 + jnp.exp(-z))
        return 0

    lax.fori_loop(0, GROUPS, reduce_body, 0)

    pltpu.sync_copy(out_v, out_hbm.at[pl.ds(base, BPW)])


def kernel(x, u_table, m_table, fc_w, fc_b):
    uidx = x[:, 0].astype(jnp.int32)
    midx = x[:, 1].astype(jnp.int32)
    ut = u_table.T
    mt = m_table.T
    fc = jnp.zeros((L,), jnp.float32)
    fc = fc.at[0].set(fc_w[0, 0]).at[1].set(fc_b[0])
    out = _emb_fwd(uidx, midx, ut, mt, fc)
    return out.reshape(BATCH, 1)
